# baseline (speedup 1.0000x reference)
import functools

import jax
import jax.numpy as jnp
import numpy as np
from jax import lax
from jax.experimental import pallas as pl
from jax.experimental.pallas import tpu as pltpu
from jax.experimental.pallas import tpu_sc as plsc

LOC_DELTA = np.float32(0.05)
MAX_INDEX = 512.0
MAX_VALUE_Z = np.float32(10.0)
MAX_LOC = np.float32((MAX_INDEX - 1.0) * 0.05)
MAGIC = np.float32(8388608.0)

B, K = 16384, 50
NC, NS, L = 2, 16, 16
NW = NC * NS
BPW = B // NW
CB = 64
NCH = BPW // CB
CKP = CB * K
NGRP = CKP // 128

_I = np.arange(CKP, dtype=np.int32)
_KP = (_I % CB) * K + _I // CB
_T2 = ((_KP // 128) * 512 + 128 + _KP % 128).astype(np.int32)


def _sc_body(in_hbm, t2_hbm, xy_hbm, z_hbm, idx_hbm,
             x0, y0, z0, x1, y1, z1, zo0, zo1, io0, io1, t2_v,
             sin0, sin1, sxy0, sxy1, szo0, szo1, sio0, sio1):
    wid = lax.axis_index("s") * NC + lax.axis_index("c")
    iota = lax.iota(jnp.int32, L)

    bufs = [(x0, y0, z0), (x1, y1, z1)]
    zos, ios = [zo0, zo1], [io0, io1]
    sins, sxys, szos, sios = [sin0, sin1], [sxy0, sxy1], [szo0, szo1], [sio0, sio1]

    def start_in(c):
        s = c % 2
        b0 = wid * BPW + c * CB
        return [
            pltpu.async_copy(in_hbm.at[p, :, pl.ds(b0, CB)], bufs[s][p], sins[s])
            for p in range(3)
        ]

    v50 = lax.iota(jnp.int32, L) * K

    def compute(c):
        s = c % 2
        xv, yv, zv = bufs[s]
        zo, io = zos[s], ios[s]
        b0 = wid * BPW + c * CB

        @plsc.parallel_loop(0, K, 1, unroll=2)
        def kbody(k):
            for j in range(CB // L):
                sl = pl.ds(16 * j, L)
                tsl = pl.ds(k * CB + 16 * j, L)
                x = jnp.minimum(xv[k, sl], MAX_LOC)
                y = jnp.minimum(yv[k, sl], MAX_LOC)
                xv[k, sl] = x
                yv[k, sl] = y
                rx = (x / LOC_DELTA + MAGIC) - MAGIC
                ry = (y / LOC_DELTA + MAGIC) - MAGIC
                rx = jnp.minimum(jnp.maximum(rx, np.float32(0.0)), np.float32(511.0))
                ry = jnp.minimum(jnp.maximum(ry, np.float32(0.0)), np.float32(511.0))
                zc = jnp.minimum(zv[k, sl], MAX_VALUE_Z)
                t1 = v50 + (k + 800 * j)
                t2 = t2_v[tsl]
                plsc.store_scatter(zo, [t1], zc)
                plsc.store_scatter(io, [t2 - 128], iota + (b0 + 16 * j))
                plsc.store_scatter(io, [t2], rx.astype(jnp.int32))
                plsc.store_scatter(io, [t2 + 128], ry.astype(jnp.int32))

    t2_h = pltpu.async_copy(t2_hbm, t2_v, sios[1])
    in_h = {0: start_in(0)}
    t2_h.wait()
    xy_h, zo_h, io_h = {}, {}, {}
    for c in range(NCH):
        s = c % 2
        b0 = wid * BPW + c * CB
        for h in in_h.pop(c):
            h.wait()
        if c + 1 < NCH:
            if c - 1 in xy_h:
                for h in xy_h.pop(c - 1):
                    h.wait()
            in_h[c + 1] = start_in(c + 1)
        if c - 2 in zo_h:
            zo_h.pop(c - 2).wait()
            io_h.pop(c - 2).wait()
        compute(c)
        xv, yv, _ = bufs[s]
        g = b0 // 128
        off = (c % 2) * CB
        xy_h[c] = [
            pltpu.async_copy(xv, xy_hbm.at[:, g, 0, pl.ds(off, CB)], sxys[s]),
            pltpu.async_copy(yv, xy_hbm.at[:, g, 1, pl.ds(off, CB)], sxys[s]),
        ]
        zo_h[c] = pltpu.async_copy(zos[s], z_hbm.at[pl.ds(b0 * K, CKP)], szos[s])
        io_h[c] = pltpu.async_copy(ios[s], idx_hbm.at[pl.ds(b0 * K * 4, CKP * 4)], sios[s])
    for hs in xy_h.values():
        for h in hs:
            h.wait()
    for h in zo_h.values():
        h.wait()
    for h in io_h.values():
        h.wait()


_sc_call = functools.partial(
    pl.kernel,
    mesh=plsc.VectorSubcoreMesh(core_axis_name="c", subcore_axis_name="s"),
    compiler_params=pltpu.CompilerParams(
        needs_layout_passes=False, use_tc_tiling_on_sc=False
    ),
    out_type=[
        jax.ShapeDtypeStruct((K, 128, 2, 128), jnp.float32),
        jax.ShapeDtypeStruct((B * K,), jnp.float32),
        jax.ShapeDtypeStruct((B * K * 4,), jnp.int32),
    ],
    scratch_types=[
        pltpu.VMEM((K, CB), jnp.float32),
        pltpu.VMEM((K, CB), jnp.float32),
        pltpu.VMEM((K, CB), jnp.float32),
        pltpu.VMEM((K, CB), jnp.float32),
        pltpu.VMEM((K, CB), jnp.float32),
        pltpu.VMEM((K, CB), jnp.float32),
        pltpu.VMEM((CKP,), jnp.float32),
        pltpu.VMEM((CKP,), jnp.float32),
        pltpu.VMEM((CKP * 4,), jnp.int32),
        pltpu.VMEM((CKP * 4,), jnp.int32),
        pltpu.VMEM((CKP,), jnp.int32),
        pltpu.SemaphoreType.DMA,
        pltpu.SemaphoreType.DMA,
        pltpu.SemaphoreType.DMA,
        pltpu.SemaphoreType.DMA,
        pltpu.SemaphoreType.DMA,
        pltpu.SemaphoreType.DMA,
        pltpu.SemaphoreType.DMA,
        pltpu.SemaphoreType.DMA,
    ],
)(_sc_body)


def kernel(inputs):
    tin = jnp.transpose(inputs, (2, 1, 0))
    xy4, z, idxf = _sc_call(tin, jnp.asarray(_T2))
    xy = xy4.transpose(1, 3, 0, 2).reshape(B, K, 2)
    idx = idxf.reshape(B * K // 128, 4, 128)[:, 0:3, :].transpose(0, 2, 1).reshape(B * K, 3)
    return (xy, z, idx)


# speedup vs baseline: 1.0130x; 1.0130x over previous
import functools

import jax
import jax.numpy as jnp
import numpy as np
from jax import lax
from jax.experimental import pallas as pl
from jax.experimental.pallas import tpu as pltpu
from jax.experimental.pallas import tpu_sc as plsc

LOC_DELTA = np.float32(0.05)
MAX_INDEX = 512.0
MAX_VALUE_Z = np.float32(10.0)
MAX_LOC = np.float32((MAX_INDEX - 1.0) * 0.05)
MAGIC = np.float32(8388608.0)

B, K = 16384, 50
NC, NS, L = 2, 16, 16
NW = NC * NS
BPW = B // NW
CB = 128
NCH = BPW // CB
CKP = CB * K
NGRP = CKP // 128

_I = np.arange(CKP, dtype=np.int32)
_KP = (_I % CB) * K + _I // CB
_T2 = ((_KP // 128) * 512 + 128 + _KP % 128).astype(np.int32)


def _sc_body(in_hbm, t2_hbm, xy_hbm, z_hbm, idx_hbm,
             x0, y0, z0, x1, y1, z1, zo0, zo1, io0, io1, t2_v,
             sin0, sin1, sxy0, sxy1, szo0, szo1, sio0, sio1):
    wid = lax.axis_index("s") * NC + lax.axis_index("c")
    iota = lax.iota(jnp.int32, L)

    bufs = [(x0, y0, z0), (x1, y1, z1)]
    zos, ios = [zo0, zo1], [io0, io1]
    sins, sxys, szos, sios = [sin0, sin1], [sxy0, sxy1], [szo0, szo1], [sio0, sio1]

    def start_in(c):
        s = c % 2
        b0 = wid * BPW + c * CB
        return [
            pltpu.async_copy(in_hbm.at[p, :, pl.ds(b0, CB)], bufs[s][p], sins[s])
            for p in range(3)
        ]

    v50 = lax.iota(jnp.int32, L) * K

    def compute(c):
        s = c % 2
        xv, yv, zv = bufs[s]
        zo, io = zos[s], ios[s]
        b0 = wid * BPW + c * CB

        @plsc.parallel_loop(0, K, 1, unroll=2)
        def kbody(k):
            for j in range(CB // L):
                sl = pl.ds(16 * j, L)
                tsl = pl.ds(k * CB + 16 * j, L)
                x = jnp.minimum(xv[k, sl], MAX_LOC)
                y = jnp.minimum(yv[k, sl], MAX_LOC)
                xv[k, sl] = x
                yv[k, sl] = y
                rx = (x / LOC_DELTA + MAGIC) - MAGIC
                ry = (y / LOC_DELTA + MAGIC) - MAGIC
                rx = jnp.minimum(jnp.maximum(rx, np.float32(0.0)), np.float32(511.0))
                ry = jnp.minimum(jnp.maximum(ry, np.float32(0.0)), np.float32(511.0))
                zc = jnp.minimum(zv[k, sl], MAX_VALUE_Z)
                t1 = v50 + (k + 800 * j)
                t2 = t2_v[tsl]
                plsc.store_scatter(zo, [t1], zc)
                plsc.store_scatter(io, [t2 - 128], iota + (b0 + 16 * j))
                plsc.store_scatter(io, [t2], rx.astype(jnp.int32))
                plsc.store_scatter(io, [t2 + 128], ry.astype(jnp.int32))

    t2_h = pltpu.async_copy(t2_hbm, t2_v, sios[1])
    in_h = {0: start_in(0)}
    t2_h.wait()
    xy_h, zo_h, io_h = {}, {}, {}
    for c in range(NCH):
        s = c % 2
        b0 = wid * BPW + c * CB
        for h in in_h.pop(c):
            h.wait()
        if c + 1 < NCH:
            if c - 1 in xy_h:
                for h in xy_h.pop(c - 1):
                    h.wait()
            in_h[c + 1] = start_in(c + 1)
        if c - 2 in zo_h:
            zo_h.pop(c - 2).wait()
            io_h.pop(c - 2).wait()
        compute(c)
        xv, yv, _ = bufs[s]
        g = b0 // 128
        xy_h[c] = [
            pltpu.async_copy(xv, xy_hbm.at[:, g, 0, :], sxys[s]),
            pltpu.async_copy(yv, xy_hbm.at[:, g, 1, :], sxys[s]),
        ]
        zo_h[c] = pltpu.async_copy(zos[s], z_hbm.at[pl.ds(b0 * K, CKP)], szos[s])
        io_h[c] = pltpu.async_copy(ios[s], idx_hbm.at[pl.ds(b0 * K * 4, CKP * 4)], sios[s])
    for hs in xy_h.values():
        for h in hs:
            h.wait()
    for h in zo_h.values():
        h.wait()
    for h in io_h.values():
        h.wait()


_sc_call = functools.partial(
    pl.kernel,
    mesh=plsc.VectorSubcoreMesh(core_axis_name="c", subcore_axis_name="s"),
    compiler_params=pltpu.CompilerParams(
        needs_layout_passes=False, use_tc_tiling_on_sc=False
    ),
    out_type=[
        jax.ShapeDtypeStruct((K, 128, 2, 128), jnp.float32),
        jax.ShapeDtypeStruct((B * K,), jnp.float32),
        jax.ShapeDtypeStruct((B * K * 4,), jnp.int32),
    ],
    scratch_types=[
        pltpu.VMEM((K, CB), jnp.float32),
        pltpu.VMEM((K, CB), jnp.float32),
        pltpu.VMEM((K, CB), jnp.float32),
        pltpu.VMEM((K, CB), jnp.float32),
        pltpu.VMEM((K, CB), jnp.float32),
        pltpu.VMEM((K, CB), jnp.float32),
        pltpu.VMEM((CKP,), jnp.float32),
        pltpu.VMEM((CKP,), jnp.float32),
        pltpu.VMEM((CKP * 4,), jnp.int32),
        pltpu.VMEM((CKP * 4,), jnp.int32),
        pltpu.VMEM((CKP,), jnp.int32),
        pltpu.SemaphoreType.DMA,
        pltpu.SemaphoreType.DMA,
        pltpu.SemaphoreType.DMA,
        pltpu.SemaphoreType.DMA,
        pltpu.SemaphoreType.DMA,
        pltpu.SemaphoreType.DMA,
        pltpu.SemaphoreType.DMA,
        pltpu.SemaphoreType.DMA,
    ],
)(_sc_body)


def kernel(inputs):
    tin = jnp.transpose(inputs, (2, 1, 0))
    xy4, z, idxf = _sc_call(tin, jnp.asarray(_T2))
    xy = xy4.transpose(1, 3, 0, 2).reshape(B, K, 2)
    idx = idxf.reshape(B * K // 128, 4, 128)[:, 0:3, :].transpose(0, 2, 1).reshape(B * K, 3)
    return (xy, z, idx)
